# Initial kernel scaffold; baseline (speedup 1.0000x reference)
import jax
import jax.numpy as jnp
from jax.experimental import pallas as pl


def kernel(x_user, x_event, edge_index, params):
    raise NotImplementedError("write your pallas kernel here")


# speedup vs baseline: 4.8241x; 4.8241x over previous
import functools

import jax
import jax.numpy as jnp
from jax import lax
from jax.experimental import pallas as pl
from jax.experimental.pallas import tpu as pltpu
from jax.experimental.pallas import tpu_sc as plsc

N_NODES = 10000
D = 128
DH = D // 2
E = 320000
NC = 2
NS = 16
CHUNK = 128
NCH = 157
EPT = NCH * CHUNK
E_PAD = NS * EPT
R = 10240
RPT = R // NS
ZCH = RPT // 128

_mesh = plsc.VectorSubcoreMesh(
    core_axis_name="c", subcore_axis_name="s", num_cores=NC, num_subcores=NS)


def _seg_body(want_counts, *refs):
    if want_counts:
        (table, srcb, dstb, z64, z16, ones16, sums_out, cnt_out,
         idx_s, idx_d, rows, zbuf, cbuf, ones_v, accum, accum_c, sem) = refs
    else:
        (table, srcb, dstb, z64, sums_out,
         idx_s, idx_d, rows, zbuf, accum, sem) = refs

    c = lax.axis_index("c")
    s = lax.axis_index("s")
    wid = c * NS + s
    row0 = s * RPT

    pltpu.sync_copy(srcb.at[wid], idx_s)
    pltpu.sync_copy(dstb.at[wid], idx_d)

    pltpu.sync_copy(z64, zbuf)
    for z in range(ZCH):
        pltpu.sync_copy(zbuf, accum.at[pl.ds(row0 + z * 128, 128)])
    if want_counts:
        pltpu.sync_copy(z16, cbuf)
        for z in range(ZCH):
            pltpu.sync_copy(cbuf, accum_c.at[pl.ds(row0 + z * 128, 128)])
        pltpu.sync_copy(ones16, ones_v)
    plsc.subcore_barrier()

    @pl.loop(0, NCH)
    def _(j):
        pltpu.async_copy(table.at[idx_s.at[j]], rows, sem).wait()
        pltpu.sync_copy(rows, accum.at[idx_d.at[j]], add=True)
        if want_counts:
            pltpu.sync_copy(ones_v, accum_c.at[idx_d.at[j]], add=True)

    plsc.subcore_barrier()

    out0 = c * R + row0
    for z in range(ZCH):
        pltpu.sync_copy(accum.at[pl.ds(row0 + z * 128, 128)], zbuf)
        pltpu.sync_copy(zbuf, sums_out.at[pl.ds(out0 + z * 128, 128)])
    if want_counts:
        @pl.when(c == 0)
        def _():
            for z in range(ZCH):
                pltpu.sync_copy(accum_c.at[pl.ds(row0 + z * 128, 128)], cbuf)
                pltpu.sync_copy(cbuf, cnt_out.at[pl.ds(row0 + z * 128, 128)])


_sc_params = pltpu.CompilerParams(use_tc_tiling_on_sc=False)

_seg_sum_counts = pl.kernel(
    functools.partial(_seg_body, True),
    out_type=(
        jax.ShapeDtypeStruct((NC * R, DH), jnp.float32),
        jax.ShapeDtypeStruct((R, 16), jnp.float32),
    ),
    mesh=_mesh,
    compiler_params=_sc_params,
    scratch_types=[
        pltpu.VMEM((NCH, CHUNK), jnp.int32),
        pltpu.VMEM((NCH, CHUNK), jnp.int32),
        pltpu.VMEM((CHUNK, DH), jnp.float32),
        pltpu.VMEM((128, DH), jnp.float32),
        pltpu.VMEM((128, 16), jnp.float32),
        pltpu.VMEM((128, 16), jnp.float32),
        pltpu.VMEM_SHARED((R, DH), jnp.float32),
        pltpu.VMEM_SHARED((R, 16), jnp.float32),
        pltpu.SemaphoreType.DMA,
    ],
)

_seg_sum = pl.kernel(
    functools.partial(_seg_body, False),
    out_type=jax.ShapeDtypeStruct((NC * R, DH), jnp.float32),
    mesh=_mesh,
    compiler_params=_sc_params,
    scratch_types=[
        pltpu.VMEM((NCH, CHUNK), jnp.int32),
        pltpu.VMEM((NCH, CHUNK), jnp.int32),
        pltpu.VMEM((CHUNK, DH), jnp.float32),
        pltpu.VMEM((128, DH), jnp.float32),
        pltpu.VMEM_SHARED((R, DH), jnp.float32),
        pltpu.SemaphoreType.DMA,
    ],
)


def _dense_body(s_ref, c_ref, x_ref, wl_ref, wr_ref, b_ref, g_ref, bt_ref,
                o_ref):
    S = jnp.concatenate(
        [s_ref[0:N_NODES, :], s_ref[R:R + N_NODES, :]], axis=1)
    cnt = c_ref[0:N_NODES, 0:1]
    agg = S / jnp.maximum(cnt, 1.0)
    xu = (jnp.dot(agg, wl_ref[...], preferred_element_type=jnp.float32)
          + jnp.dot(x_ref[...], wr_ref[...], preferred_element_type=jnp.float32)
          + b_ref[...])
    m = jnp.mean(xu, axis=0, keepdims=True)
    d = xu - m
    v = jnp.mean(d * d, axis=0, keepdims=True)
    y = d * lax.rsqrt(v + 1e-5) * g_ref[...] + bt_ref[...]
    o_ref[...] = jnp.maximum(y, 0.0)


_dense = pl.pallas_call(
    _dense_body,
    out_shape=jax.ShapeDtypeStruct((N_NODES, D), jnp.float32),
)


def _as_blocks(idx, fill, bias):
    pad = jnp.full((E_PAD - E,), fill, jnp.int32)
    blk = jnp.concatenate([idx, pad]).reshape(1, NS, NCH, CHUNK)
    return jnp.concatenate([blk, blk + bias], axis=0).reshape(
        NC * NS, NCH, CHUNK)


def _split_cols(x):
    return jnp.concatenate([x[:, :DH], x[:, DH:]], axis=0)


def kernel(x_user, x_event, edge_index, params):
    u = edge_index[0].astype(jnp.int32)
    e = edge_index[1].astype(jnp.int32)
    src_u = _as_blocks(e, 0, N_NODES)
    dst_u = _as_blocks(u, R - 1, 0)
    src_e = _as_blocks(u, 0, N_NODES)
    dst_e = _as_blocks(e, R - 1, 0)

    z64 = jnp.zeros((128, DH), jnp.float32)
    z16 = jnp.zeros((128, 16), jnp.float32)
    ones16 = jnp.ones((128, 16), jnp.float32)

    def dense(S, C, x, side, i):
        return _dense(S, C, x,
                      params['Wl_%s%d' % (side, i)].T,
                      params['Wr_%s%d' % (side, i)].T,
                      params['bl_%s%d' % (side, i)].reshape(1, D),
                      params['gamma_%s%d' % (side, i)].reshape(1, D),
                      params['beta_%s%d' % (side, i)].reshape(1, D))

    Su, Cu = _seg_sum_counts(_split_cols(x_event), src_u, dst_u,
                             z64, z16, ones16)
    x_user = dense(Su, Cu, x_user, 'u', 0)
    Se, Ce = _seg_sum_counts(_split_cols(x_user), src_e, dst_e,
                             z64, z16, ones16)
    x_event = dense(Se, Ce, x_event, 'e', 0)

    Su2 = _seg_sum(_split_cols(x_event), src_u, dst_u, z64)
    x_user = dense(Su2, Cu, x_user, 'u', 1)
    Se2 = _seg_sum(_split_cols(x_user), src_e, dst_e, z64)
    x_event = dense(Se2, Ce, x_event, 'e', 1)
    return x_user, x_event
